# baseline (speedup 1.0000x reference)
import dataclasses
import functools

import jax
import jax.numpy as jnp
from jax.experimental import pallas as pl
from jax.experimental.pallas import tpu as pltpu
from jax.experimental.pallas import tpu_sc as plsc

_B, _T, _C = 4, 2048, 1024
_H, _F = 16, 64
_C2 = 2048
_KS = 25
_PAD = (_KS - 1) // 2
_TOPK = 8
_RT = 256
_PT = 512
_CT = 512

_f32 = jnp.float32
_bf16 = jnp.bfloat16


def _proj_body(x_ref, wq_ref, bq_ref, wk_ref, bk_ref, q_ref, k_ref):
    x = x_ref[...].astype(_bf16)
    q = jnp.dot(x, wq_ref[...], preferred_element_type=_f32) + bq_ref[...]
    k = jnp.dot(x, wk_ref[...], preferred_element_type=_f32) + bk_ref[...]
    q_ref[...] = q.astype(_bf16)
    k_ref[...] = k.astype(_bf16)


def _project(xf, wq, bq, wk, bk):
    n = _B * _T
    return pl.pallas_call(
        _proj_body,
        grid=(n // _PT,),
        in_specs=[
            pl.BlockSpec((_PT, _C), lambda i: (i, 0)),
            pl.BlockSpec((_C, _C), lambda i: (0, 0)),
            pl.BlockSpec((1, _C), lambda i: (0, 0)),
            pl.BlockSpec((_C, _C), lambda i: (0, 0)),
            pl.BlockSpec((1, _C), lambda i: (0, 0)),
        ],
        out_specs=[
            pl.BlockSpec((_PT, _C), lambda i: (i, 0)),
            pl.BlockSpec((_PT, _C), lambda i: (i, 0)),
        ],
        out_shape=[
            jax.ShapeDtypeStruct((n, _C), _bf16),
            jax.ShapeDtypeStruct((n, _C), _bf16),
        ],
        compiler_params=pltpu.CompilerParams(
            dimension_semantics=("parallel",),
        ),
    )(xf, wq, bq, wk, bk)


def _corr_body(q_ref, k_ref, p_ref):
    t = pl.program_id(1)
    g = jax.lax.dot_general(
        q_ref[0], k_ref[0], (((1,), (1,)), ((), ())),
        preferred_element_type=_f32,
    )

    y = g
    shift = _RT // 2
    while shift >= 1:
        y = y[: shift] + jnp.roll(y[shift: 2 * shift], -shift, axis=1)
        shift //= 2
    r0 = t * _RT
    for s in (256, 512, 1024):
        y = jnp.where((r0 & s) != 0, jnp.roll(y, -s, axis=1), y)

    @pl.when(t == 0)
    def _():
        p_ref[...] = jnp.zeros_like(p_ref)

    p_ref[0] += y


def _autocorr(q3, k3):
    return pl.pallas_call(
        _corr_body,
        grid=(_B, _T // _RT),
        in_specs=[
            pl.BlockSpec((1, _RT, _C), lambda b, t: (b, t, 0)),
            pl.BlockSpec((1, _T, _C), lambda b, t: (b, 0, 0)),
        ],
        out_specs=pl.BlockSpec((1, 1, _T), lambda b, t: (b, 0, 0)),
        out_shape=jax.ShapeDtypeStruct((_B, 1, _T), _f32),
        compiler_params=pltpu.CompilerParams(
            dimension_semantics=("arbitrary", "arbitrary"),
        ),
    )(q3, k3)


def _sc_select(p2, k8f):
    mesh = plsc.VectorSubcoreMesh(core_axis_name="c", subcore_axis_name="s")
    neg = jnp.float32(-jnp.inf)
    nchunk = 32

    cp = pltpu.CompilerParams()
    if "needs_layout_passes" in pltpu.CompilerParams.__dataclass_fields__:
        cp = dataclasses.replace(cp, needs_layout_passes=False)

    @functools.partial(
        pl.kernel,
        out_type=jax.ShapeDtypeStruct((_B * _TOPK, _H * _F), _f32),
        mesh=mesh,
        compiler_params=cp,
        scratch_types=[
            pltpu.VMEM((512,), _f32),
            pltpu.VMEM((16,), _f32),
            pltpu.VMEM_SHARED((16, 16), _f32),
            pltpu.VMEM((16, 16), _f32),
            pltpu.VMEM((_B, 16), _f32),
            pltpu.VMEM((_B, 16), _f32),
            pltpu.VMEM((16,), jnp.int32),
            pltpu.VMEM((_H * _F,), _f32),
            pltpu.VMEM((_H * _F,), _f32),
        ],
    )
    def sel(p_hbm, k8_hbm, nr_hbm, chunk_v, top8_v, shared_v, cand_v, w_v,
            wsel_v, ord_v, row_v, nrow_v):
        c = jax.lax.axis_index("c")
        s = jax.lax.axis_index("s")
        negv = jnp.full((16,), neg, _f32)
        ii16 = jax.lax.iota(jnp.int32, 16)

        b1 = jax.lax.div(s, 4)
        ch = jax.lax.rem(s, 4)
        pltpu.sync_copy(p_hbm.at[b1, pl.ds(pl.multiple_of(ch * 512, 512), 512)],
                        chunk_v)
        top8_v[...] = negv
        for r in range(_TOPK):
            vm = chunk_v[pl.ds(0, 16)]
            for i in range(1, nchunk):
                vm = jnp.maximum(vm, chunk_v[pl.ds(16 * i, 16)])
            m = jnp.max(vm)
            msp = jnp.full((16,), m, _f32)
            top8_v[...] = jnp.where(ii16 == r, msp, top8_v[...])
            for i in range(nchunk):
                rg = chunk_v[pl.ds(16 * i, 16)]
                chunk_v[pl.ds(16 * i, 16)] = jnp.where(rg == msp, negv, rg)
        pltpu.sync_copy(top8_v, shared_v.at[s])
        plsc.subcore_barrier()

        pltpu.sync_copy(shared_v, cand_v)
        for b in range(_B):
            for j in range(4):
                chunk_v[pl.ds(16 * j, 16)] = cand_v[b * 4 + j, :]
            wrow = negv
            for r in range(_TOPK):
                vm = chunk_v[pl.ds(0, 16)]
                for j in range(1, 4):
                    vm = jnp.maximum(vm, chunk_v[pl.ds(16 * j, 16)])
                m = jnp.max(vm)
                msp = jnp.full((16,), m, _f32)
                wrow = jnp.where(ii16 == r, msp * (1.0 / (_H * _F)), wrow)
                for j in range(4):
                    rg = chunk_v[pl.ds(16 * j, 16)]
                    chunk_v[pl.ds(16 * j, 16)] = jnp.where(rg == msp, negv, rg)
            w_v[b, :] = wrow
        wm = (w_v[0, :] + w_v[1, :] + w_v[2, :] + w_v[3, :]) * 0.25
        top8_v[...] = wm
        for r in range(_TOPK):
            tv = top8_v[...]
            best = tv[0]
            bi = jnp.int32(0)
            for j in range(1, _TOPK):
                v = tv[j]
                better = v > best
                best = jnp.where(better, v, best)
                bi = jnp.where(better, jnp.int32(j), bi)
            bisp = jnp.full((16,), bi, jnp.int32)
            ord_v[...] = jnp.where(ii16 == r, bisp, ord_v[...])
            top8_v[...] = jnp.where(ii16 == bisp, negv, tv)
        ovec = ord_v[...]
        for b in range(_B):
            wrow = w_v[b, :]
            sm = negv
            for r in range(_TOPK):
                pick = jnp.sum(jnp.where(ii16 == jnp.full((16,), ovec[r],
                                                          jnp.int32),
                                         wrow, jnp.zeros((16,), _f32)))
                sm = jnp.where(ii16 == r, jnp.full((16,), pick, _f32), sm)
            e = jnp.exp(sm - jnp.full((16,), jnp.max(sm), _f32))
            wsel_v[b, :] = e / jnp.full((16,), jnp.sum(e), _f32)

        rrow = c * 16 + s
        bb = jax.lax.div(rrow, _TOPK)
        idx = jax.lax.rem(rrow, _TOPK)
        idxsp = jnp.full((16,), idx, jnp.int32)
        o = jnp.sum(jnp.where(ii16 == idxsp, ovec, jnp.zeros((16,), jnp.int32)))
        wt = jnp.float32(0.0)
        for b in range(_B):
            wb = jnp.sum(jnp.where(ii16 == idxsp, wsel_v[b, :],
                                   jnp.zeros((16,), _f32)))
            wt = jnp.where(bb == b, wb, wt)
        pltpu.sync_copy(k8_hbm.at[rrow], row_v)
        osp = jnp.full((16,), o, jnp.int32)
        for c0 in range(0, _H * _F, 16):
            base = ii16 + c0
            f = jax.lax.bitwise_and(base, _F - 1)
            h64 = base - f
            src = h64 + jax.lax.bitwise_and(f + osp, _F - 1)
            g = plsc.load_gather(row_v, [src])
            nrow_v[pl.ds(c0, 16)] = g * wt
        pltpu.sync_copy(nrow_v, nr_hbm.at[rrow])

    return sel(p2, k8f)


def _outproj_body(nr_ref, kk_ref, x_ref, wo_ref, bo_ref, x1_ref):
    i = pl.program_id(0)
    tiles_per_batch = _T // _PT
    b = i // tiles_per_batch
    vin = kk_ref[...]
    top = jnp.where(i % tiles_per_batch == 0,
                    nr_ref[pl.ds(pl.multiple_of(b * _TOPK, _TOPK), _TOPK)]
                    .astype(_bf16), vin[: _TOPK])
    vin = jnp.concatenate([top, vin[_TOPK:]], axis=0)
    out = jnp.dot(vin, wo_ref[...], preferred_element_type=_f32) + bo_ref[...]
    x1_ref[...] = x_ref[...] + out


def _outproj(nr, kk, xf, wo, bo):
    n = _B * _T
    return pl.pallas_call(
        _outproj_body,
        grid=(n // _PT,),
        in_specs=[
            pl.BlockSpec((_B * _TOPK, _H * _F), lambda i: (0, 0)),
            pl.BlockSpec((_PT, _C), lambda i: (i, 0)),
            pl.BlockSpec((_PT, _C), lambda i: (i, 0)),
            pl.BlockSpec((_C, _C), lambda i: (0, 0)),
            pl.BlockSpec((1, _C), lambda i: (0, 0)),
        ],
        out_specs=pl.BlockSpec((_PT, _C), lambda i: (i, 0)),
        out_shape=jax.ShapeDtypeStruct((n, _C), _f32),
        compiler_params=pltpu.CompilerParams(
            dimension_semantics=("arbitrary",),
        ),
    )(nr, kk, xf, wo, bo)


def _decomp_body(x_ref, y_ref, *, out_dtype):
    x = x_ref[0]
    front = jnp.broadcast_to(x[0:1], (_PAD, _C))
    end = jnp.broadcast_to(x[_T - 1: _T], (_PAD, _C))
    xp = jnp.concatenate([front, x, end], axis=0)
    p2 = xp[: _T + 23] + xp[1: _T + 24]
    p4 = p2[: _T + 21] + p2[2: _T + 23]
    p8 = p4[: _T + 17] + p4[4: _T + 21]
    w24 = p8[: _T] + p8[8: _T + 8] + p8[16: _T + 16]
    mm = (w24 + xp[24: _T + 24]) * (1.0 / _KS)
    y_ref[0] = (x - mm).astype(out_dtype)


def _decomp(x3, out_dtype):
    return pl.pallas_call(
        functools.partial(_decomp_body, out_dtype=out_dtype),
        grid=(_B,),
        in_specs=[pl.BlockSpec((1, _T, _C), lambda b: (b, 0, 0))],
        out_specs=pl.BlockSpec((1, _T, _C), lambda b: (b, 0, 0)),
        out_shape=jax.ShapeDtypeStruct((_B, _T, _C), out_dtype),
        compiler_params=pltpu.CompilerParams(
            dimension_semantics=("parallel",),
        ),
    )(x3)


def _conv_body(y_ref, w_ref, aux_ref, o_ref, *, cin, relu):
    t = pl.program_id(1)
    t0 = t * _CT
    cur = y_ref[0, pl.ds(pl.multiple_of(t0, _CT), _CT)]
    zrow = jnp.zeros((1, cin), _bf16)
    pbase = pl.multiple_of(jnp.maximum(t0 - 8, 0), 8)
    prev = jnp.where(t == 0, zrow, y_ref[0, pl.ds(pbase, 8)][7:8])
    nbase = pl.multiple_of(jnp.minimum(t0 + _CT, _T - 8), 8)
    nxt = jnp.where(t0 + _CT >= _T, zrow, y_ref[0, pl.ds(nbase, 8)][0:1])
    ym1 = jnp.concatenate([prev, cur[:-1]], axis=0)
    yp1 = jnp.concatenate([cur[1:], nxt], axis=0)
    nt = (((1,), (1,)), ((), ()))
    acc = jax.lax.dot_general(ym1, w_ref[0], nt, preferred_element_type=_f32)
    acc = acc + jax.lax.dot_general(cur, w_ref[1], nt,
                                    preferred_element_type=_f32)
    acc = acc + jax.lax.dot_general(yp1, w_ref[2], nt,
                                    preferred_element_type=_f32)
    if relu:
        o_ref[0] = jnp.maximum(acc, 0.0).astype(o_ref.dtype)
    else:
        o_ref[0] = (aux_ref[0] + acc).astype(o_ref.dtype)


def _conv(y3, w, aux, cin, cout, relu, out_dtype):
    body = functools.partial(_conv_body, cin=cin, relu=relu)
    in_specs = [
        pl.BlockSpec((1, _T, cin), lambda b, t: (b, 0, 0)),
        pl.BlockSpec((3, cout, cin), lambda b, t: (0, 0, 0)),
        pl.BlockSpec((1, _CT, cout), lambda b, t: (b, t, 0)),
    ]
    if aux is None:
        aux = jnp.zeros((1, _CT, cout), out_dtype)
        in_specs[2] = pl.BlockSpec((1, _CT, cout), lambda b, t: (0, 0, 0))
    return pl.pallas_call(
        body,
        grid=(_B, _T // _CT),
        in_specs=in_specs,
        out_specs=pl.BlockSpec((1, _CT, cout), lambda b, t: (b, t, 0)),
        out_shape=jax.ShapeDtypeStruct((_B, _T, cout), out_dtype),
        compiler_params=pltpu.CompilerParams(
            dimension_semantics=("parallel", "parallel"),
        ),
    )(y3, w, aux)


def kernel(X, Wq, bq, Wk, bk, Wo, bo, Wc1, Wc2):
    xf = X.reshape(_B * _T, _C)
    wq = Wq.astype(_bf16)
    wk = Wk.astype(_bf16)
    wo = Wo.astype(_bf16)
    w1 = jnp.transpose(Wc1.astype(_bf16), (2, 0, 1))
    w2 = jnp.transpose(Wc2.astype(_bf16), (2, 0, 1))

    q, kk = _project(xf, wq, bq.reshape(1, _C), wk, bk.reshape(1, _C))

    q3 = q.reshape(_B, _T, _C)
    k3 = kk.reshape(_B, _T, _C)
    p = _autocorr(q3, k3)

    k8 = k3[:, : _TOPK, :].reshape(_B * _TOPK, _C).astype(_f32)
    nr = _sc_select(p.reshape(_B, _T), k8)

    x1 = _outproj(nr, kk, xf, wo, bo.reshape(1, _C))
    x13 = x1.reshape(_B, _T, _C)

    y = _decomp(x13, _bf16)
    h = _conv(y, w1, None, _C, _C2, True, _bf16)
    x2 = _conv(h, w2, x13, _C2, _C, False, _f32)
    res = _decomp(x2, _f32)
    return res
